# baseline (speedup 1.0000x reference)
import jax
import jax.numpy as jnp
from jax import lax
from jax.experimental import pallas as pl
from jax.experimental.pallas import tpu as pltpu
from jax.experimental.pallas import tpu_sc as plsc

N_NODES = 10000
D_IN = 256
D_HID = 256
N_GRAPHS = 64

NB = 5
R = N_NODES // NB

NW = 32
GP_STEP = R // 16
GP_PAD = 128
NGROUPS = NB * GP_STEP
NPHYS = NB * GP_PAD
WIN = 32
NEG = -3.0e38


def _elu(v):
    return jnp.where(v > 0, v, jnp.exp(jnp.minimum(v, 0.0)) - 1.0)


def _mm_body(h_ref, w_ref, b_ref, batch_ref, x_ref, gmax_ref, sums_ref,
             counts_ref):
    i = pl.program_id(0)
    x = lax.dot_general(h_ref[...], w_ref[...], (((1,), (0,)), ((), ())),
                        precision=lax.Precision.DEFAULT,
                        preferred_element_type=jnp.float32)
    x = _elu(x + b_ref[...])
    x_ref[...] = x
    gm = jnp.max(x.reshape(GP_STEP, 16, D_HID), axis=1)
    gmax_ref[0] = jnp.concatenate(
        [gm, jnp.full((GP_PAD - GP_STEP, D_HID), NEG, jnp.float32)], axis=0)
    ids = batch_ref[0, 0, :]
    seg = lax.broadcasted_iota(jnp.int32, (N_GRAPHS, R), 0)
    oh = (ids[None, :] == seg).astype(jnp.float32)
    part_sums = lax.dot_general(oh, x, (((1,), (0,)), ((), ())),
                                precision=lax.Precision.DEFAULT,
                                preferred_element_type=jnp.float32)
    part_counts = jnp.broadcast_to(jnp.sum(oh, axis=1)[:, None],
                                   (N_GRAPHS, 128))

    @pl.when(i == 0)
    def _():
        sums_ref[...] = jnp.zeros_like(sums_ref)
        counts_ref[...] = jnp.zeros_like(counts_ref)

    sums_ref[...] += part_sums
    counts_ref[...] += part_counts


def _run_mm(h, W, b2d, batch3):
    return pl.pallas_call(
        _mm_body,
        grid=(NB,),
        in_specs=[
            pl.BlockSpec((R, D_IN), lambda i: (i, 0)),
            pl.BlockSpec((D_IN, D_HID), lambda i: (0, 0)),
            pl.BlockSpec((1, D_HID), lambda i: (0, 0)),
            pl.BlockSpec((1, 1, R), lambda i: (i, 0, 0)),
        ],
        out_specs=[
            pl.BlockSpec((R, D_HID), lambda i: (i, 0)),
            pl.BlockSpec((1, GP_PAD, D_HID), lambda i: (i, 0, 0)),
            pl.BlockSpec((N_GRAPHS, D_HID), lambda i: (0, 0)),
            pl.BlockSpec((N_GRAPHS, 128), lambda i: (0, 0)),
        ],
        out_shape=[
            jax.ShapeDtypeStruct((N_NODES, D_HID), jnp.float32),
            jax.ShapeDtypeStruct((NB, GP_PAD, D_HID), jnp.float32),
            jax.ShapeDtypeStruct((N_GRAPHS, D_HID), jnp.float32),
            jax.ShapeDtypeStruct((N_GRAPHS, 128), jnp.float32),
        ],
    )(h, W, b2d, batch3)


def _sc_max_body(gmax_hbm, x_hbm, batch_hbm, out_hbm, gbuf, ids_buf, fx,
                 table):
    w = lax.axis_index("s") * 2 + lax.axis_index("c")
    raw = w * (NPHYS // NW)
    offset = pl.multiple_of(jnp.minimum(raw - raw % 8, NPHYS - WIN), 8)
    pltpu.sync_copy(gmax_hbm.at[pl.ds(offset, WIN)], gbuf)
    gmin = (offset // GP_PAD) * GP_STEP + offset % GP_PAD
    idstart = pl.multiple_of(jnp.minimum(gmin * 16, N_NODES - WIN * 16), 16)
    pltpu.sync_copy(batch_hbm.at[pl.ds(idstart, WIN * 16)], ids_buf)

    neg = jnp.full((16,), NEG, jnp.float32)

    def init_row(r, carry):
        for k in range(16):
            table[r, pl.ds(k * 16, 16)] = neg
        return carry

    lax.fori_loop(0, N_GRAPHS, init_row, 0)

    def flush(seg, vals):
        for k in range(16):
            cur = table[seg, pl.ds(k * 16, 16)]
            table[seg, pl.ds(k * 16, 16)] = jnp.maximum(cur, vals[k])

    def row_scalars(lp, prev_hint):
        p = offset + lp
        valid = (p % GP_PAD) < GP_STEP
        g = (p // GP_PAD) * GP_STEP + p % GP_PAD
        gl = jnp.where(valid,
                       jnp.minimum(g - (idstart // 16), WIN - 1), 0)
        ids16 = ids_buf[pl.ds(gl * 16, 16)]
        id_lo = jnp.where(valid, ids16[0], prev_hint)
        id_hi = jnp.where(valid, ids16[15], prev_hint)
        return id_lo, id_hi, g

    def row_body(lp, carry):
        prev, id_lo, id_hi, g = carry[:4]
        acc = carry[4:]
        rows = tuple(gbuf[lp, pl.ds(k * 16, 16)] for k in range(16))

        mixedp = id_lo != id_hi
        newseg = jnp.logical_or(id_lo != prev, mixedp)

        @pl.when(newseg)
        def _():
            flush(prev, acc)

        @pl.when(mixedp)
        def _():
            pltpu.sync_copy(x_hbm.at[pl.ds(pl.multiple_of(g * 16, 16), 16)],
                            fx)
            gl = g - (idstart // 16)
            ids16 = ids_buf[pl.ds(gl * 16, 16)]
            for j in range(16):
                idj = ids16[j]
                for k in range(16):
                    row = fx[j, pl.ds(k * 16, 16)]
                    cur = table[idj, pl.ds(k * 16, 16)]
                    table[idj, pl.ds(k * 16, 16)] = jnp.maximum(cur, row)

        new_acc = tuple(
            jnp.where(mixedp, neg,
                      jnp.where(newseg, r, jnp.maximum(a, r)))
            for a, r in zip(acc, rows))
        new_prev = jnp.where(mixedp, id_hi, id_lo)
        nid_lo, nid_hi, ng = row_scalars(lp + 1, new_prev)
        return (new_prev, nid_lo, nid_hi, ng) + new_acc

    lp0 = raw - offset
    prev0 = jnp.array(0, jnp.int32)
    lo0, hi0, g0 = row_scalars(lp0, prev0)
    init = (prev0, lo0, hi0, g0) + tuple(neg for _ in range(16))
    final = lax.fori_loop(lp0, lp0 + NPHYS // NW, row_body, init)
    flush(final[0], final[4:])
    pltpu.sync_copy(table, out_hbm.at[w])


def _run_sc_max(gmax, x, batch):
    kern = pl.kernel(
        _sc_max_body,
        out_type=jax.ShapeDtypeStruct((NW, N_GRAPHS, D_HID), jnp.float32),
        mesh=plsc.VectorSubcoreMesh(core_axis_name="c", subcore_axis_name="s"),
        scratch_types=[
            pltpu.VMEM((WIN, D_HID), jnp.float32),
            pltpu.VMEM((WIN * 16,), jnp.int32),
            pltpu.VMEM((16, D_HID), jnp.float32),
            pltpu.VMEM((N_GRAPHS, D_HID), jnp.float32),
        ],
    )
    return kern(gmax, x, batch)


def _out_body(h_ref, batch_ref, maxp_ref, sums_ref, counts_ref, cw_ref,
              cb_ref, out_ref):
    cnt = counts_ref[:, 0:1]
    maxc = jnp.max(maxp_ref[...], axis=0)
    mean = sums_ref[...] / jnp.maximum(cnt, 1.0)
    cw0 = cw_ref[0, 0, 0]
    cw1 = cw_ref[0, 1, 0]
    cb = cb_ref[0]
    pert = _elu(cw0 * maxc + cw1 * mean + cb)
    pert = jnp.where(cnt > 0, pert, 0.0)
    ids = batch_ref[0, 0, :]
    oh = (ids[:, None] == lax.broadcasted_iota(jnp.int32, (R, N_GRAPHS), 1)
          ).astype(jnp.float32)
    out_ref[...] = h_ref[...] + lax.dot_general(
        oh, pert, (((1,), (0,)), ((), ())),
        precision=lax.Precision.DEFAULT,
        preferred_element_type=jnp.float32)


def _run_out(h, batch3, maxp, sums, counts, conv_w, conv_b):
    return pl.pallas_call(
        _out_body,
        grid=(NB,),
        in_specs=[
            pl.BlockSpec((R, D_IN), lambda i: (i, 0)),
            pl.BlockSpec((1, 1, R), lambda i: (i, 0, 0)),
            pl.BlockSpec((NW, N_GRAPHS, D_HID), lambda i: (0, 0, 0)),
            pl.BlockSpec((N_GRAPHS, D_HID), lambda i: (0, 0)),
            pl.BlockSpec((N_GRAPHS, 128), lambda i: (0, 0)),
            pl.BlockSpec(memory_space=pltpu.SMEM),
            pl.BlockSpec(memory_space=pltpu.SMEM),
        ],
        out_specs=pl.BlockSpec((R, D_IN), lambda i: (i, 0)),
        out_shape=jax.ShapeDtypeStruct((N_NODES, D_IN), jnp.float32),
    )(h, batch3, maxp, sums, counts, conv_w, conv_b)


def kernel(h, edge_index, batch, W, b, conv_w, conv_b):
    batch3 = batch.reshape(NB, 1, R)
    b2d = b.reshape(1, D_HID)
    x, gmax, sums, counts = _run_mm(h, W, b2d, batch3)
    maxp = _run_sc_max(gmax.reshape(NPHYS, D_HID), x, batch)
    h_out = _run_out(h, batch3, maxp, sums, counts, conv_w, conv_b)
    return (h_out, edge_index, batch)


# speedup vs baseline: 1.0049x; 1.0049x over previous
import jax
import jax.numpy as jnp
from jax import lax
from jax.experimental import pallas as pl
from jax.experimental.pallas import tpu as pltpu
from jax.experimental.pallas import tpu_sc as plsc

N_NODES = 10000
D_IN = 256
D_HID = 256
N_GRAPHS = 64

NB = 5
R = N_NODES // NB

NW = 32
GP_STEP = R // 16
GP_PAD = 128
NGROUPS = NB * GP_STEP
NPHYS = NB * GP_PAD
WIN = 32
NEG = -3.0e38


def _elu(v):
    return jnp.where(v > 0, v, jnp.exp(jnp.minimum(v, 0.0)) - 1.0)


def _mm_body(h_ref, w_ref, b_ref, batch_ref, x_ref, gmax_ref, sums_ref,
             counts_ref):
    i = pl.program_id(0)
    x = lax.dot_general(h_ref[...], w_ref[...], (((1,), (0,)), ((), ())),
                        precision=lax.Precision.DEFAULT,
                        preferred_element_type=jnp.float32)
    x = _elu(x + b_ref[...])
    x_ref[...] = x
    gm = jnp.max(x.reshape(GP_STEP, 16, D_HID), axis=1)
    gmax_ref[0] = jnp.concatenate(
        [gm, jnp.full((GP_PAD - GP_STEP, D_HID), NEG, jnp.float32)], axis=0)
    ids = batch_ref[0, 0, :]
    seg = lax.broadcasted_iota(jnp.int32, (N_GRAPHS, R), 0)
    oh = (ids[None, :] == seg).astype(jnp.float32)
    part_sums = lax.dot_general(oh, x, (((1,), (0,)), ((), ())),
                                precision=lax.Precision.DEFAULT,
                                preferred_element_type=jnp.float32)
    part_counts = jnp.broadcast_to(jnp.sum(oh, axis=1)[:, None],
                                   (N_GRAPHS, 128))

    @pl.when(i == 0)
    def _():
        sums_ref[...] = jnp.zeros_like(sums_ref)
        counts_ref[...] = jnp.zeros_like(counts_ref)

    sums_ref[...] += part_sums
    counts_ref[...] += part_counts


def _run_mm(h, W, b2d, batch3):
    return pl.pallas_call(
        _mm_body,
        grid=(NB,),
        in_specs=[
            pl.BlockSpec((R, D_IN), lambda i: (i, 0)),
            pl.BlockSpec((D_IN, D_HID), lambda i: (0, 0)),
            pl.BlockSpec((1, D_HID), lambda i: (0, 0)),
            pl.BlockSpec((1, 1, R), lambda i: (i, 0, 0)),
        ],
        out_specs=[
            pl.BlockSpec((R, D_HID), lambda i: (i, 0)),
            pl.BlockSpec((1, GP_PAD, D_HID), lambda i: (i, 0, 0)),
            pl.BlockSpec((N_GRAPHS, D_HID), lambda i: (0, 0)),
            pl.BlockSpec((N_GRAPHS, 128), lambda i: (0, 0)),
        ],
        out_shape=[
            jax.ShapeDtypeStruct((N_NODES, D_HID), jnp.float32),
            jax.ShapeDtypeStruct((NB, GP_PAD, D_HID), jnp.float32),
            jax.ShapeDtypeStruct((N_GRAPHS, D_HID), jnp.float32),
            jax.ShapeDtypeStruct((N_GRAPHS, 128), jnp.float32),
        ],
    )(h, W, b2d, batch3)


def _sc_max_body(gmax_hbm, x_hbm, batch_hbm, out_hbm, gbuf, ids_buf, fx,
                 table):
    w = lax.axis_index("s") * 2 + lax.axis_index("c")
    raw = w * (NPHYS // NW)
    offset = pl.multiple_of(jnp.minimum(raw - raw % 8, NPHYS - WIN), 8)
    pltpu.sync_copy(gmax_hbm.at[pl.ds(offset, WIN)], gbuf)
    gmin = (offset // GP_PAD) * GP_STEP + offset % GP_PAD
    idstart = pl.multiple_of(jnp.minimum(gmin * 16, N_NODES - WIN * 16), 16)
    pltpu.sync_copy(batch_hbm.at[pl.ds(idstart, WIN * 16)], ids_buf)

    neg = jnp.full((16,), NEG, jnp.float32)

    def init_row(r, carry):
        for k in range(16):
            table[r, pl.ds(k * 16, 16)] = neg
        return carry

    lax.fori_loop(0, N_GRAPHS, init_row, 0)

    def flush(seg, vals):
        for k in range(16):
            cur = table[seg, pl.ds(k * 16, 16)]
            table[seg, pl.ds(k * 16, 16)] = jnp.maximum(cur, vals[k])

    def row_body(lp, carry):
        prev = carry[0]
        acc = carry[1:]
        p = offset + lp
        valid = (p % GP_PAD) < GP_STEP
        g = (p // GP_PAD) * GP_STEP + p % GP_PAD
        gl = jnp.where(valid, g - (idstart // 16), 0)
        ids16 = ids_buf[pl.ds(gl * 16, 16)]
        id_lo = jnp.where(valid, ids16[0], prev)
        id_hi = jnp.where(valid, ids16[15], prev)
        rows = tuple(gbuf[lp, pl.ds(k * 16, 16)] for k in range(16))

        mixedp = id_lo != id_hi
        newseg = jnp.logical_or(id_lo != prev, mixedp)

        @pl.when(newseg)
        def _():
            flush(prev, acc)

        @pl.when(mixedp)
        def _():
            pltpu.sync_copy(x_hbm.at[pl.ds(pl.multiple_of(g * 16, 16), 16)],
                            fx)
            for j in range(16):
                idj = ids16[j]
                for k in range(16):
                    row = fx[j, pl.ds(k * 16, 16)]
                    cur = table[idj, pl.ds(k * 16, 16)]
                    table[idj, pl.ds(k * 16, 16)] = jnp.maximum(cur, row)

        new_acc = tuple(
            jnp.where(mixedp, neg,
                      jnp.where(newseg, r, jnp.maximum(a, r)))
            for a, r in zip(acc, rows))
        new_prev = jnp.where(mixedp, id_hi, id_lo)
        return (new_prev,) + new_acc

    init = (jnp.array(0, jnp.int32),) + tuple(neg for _ in range(16))
    lp0 = raw - offset
    final = lax.fori_loop(lp0, lp0 + NPHYS // NW, row_body, init)
    flush(final[0], final[1:])
    pltpu.sync_copy(table, out_hbm.at[w])


def _run_sc_max(gmax, x, batch):
    kern = pl.kernel(
        _sc_max_body,
        out_type=jax.ShapeDtypeStruct((NW, N_GRAPHS, D_HID), jnp.float32),
        mesh=plsc.VectorSubcoreMesh(core_axis_name="c", subcore_axis_name="s"),
        scratch_types=[
            pltpu.VMEM((WIN, D_HID), jnp.float32),
            pltpu.VMEM((WIN * 16,), jnp.int32),
            pltpu.VMEM((16, D_HID), jnp.float32),
            pltpu.VMEM((N_GRAPHS, D_HID), jnp.float32),
        ],
    )
    return kern(gmax, x, batch)


def _out_body(h_ref, batch_ref, maxp_ref, sums_ref, counts_ref, cw_ref,
              cb_ref, out_ref):
    cnt = counts_ref[:, 0:1]
    maxc = jnp.max(maxp_ref[...], axis=0)
    mean = sums_ref[...] / jnp.maximum(cnt, 1.0)
    cw0 = cw_ref[0, 0, 0]
    cw1 = cw_ref[0, 1, 0]
    cb = cb_ref[0]
    pert = _elu(cw0 * maxc + cw1 * mean + cb)
    pert = jnp.where(cnt > 0, pert, 0.0)
    ids = batch_ref[0, 0, :]
    oh = (ids[:, None] == lax.broadcasted_iota(jnp.int32, (R, N_GRAPHS), 1)
          ).astype(jnp.float32)
    out_ref[...] = h_ref[...] + lax.dot_general(
        oh, pert, (((1,), (0,)), ((), ())),
        precision=lax.Precision.DEFAULT,
        preferred_element_type=jnp.float32)


def _run_out(h, batch3, maxp, sums, counts, conv_w, conv_b):
    return pl.pallas_call(
        _out_body,
        grid=(NB,),
        in_specs=[
            pl.BlockSpec((R, D_IN), lambda i: (i, 0)),
            pl.BlockSpec((1, 1, R), lambda i: (i, 0, 0)),
            pl.BlockSpec((NW, N_GRAPHS, D_HID), lambda i: (0, 0, 0)),
            pl.BlockSpec((N_GRAPHS, D_HID), lambda i: (0, 0)),
            pl.BlockSpec((N_GRAPHS, 128), lambda i: (0, 0)),
            pl.BlockSpec(memory_space=pltpu.SMEM),
            pl.BlockSpec(memory_space=pltpu.SMEM),
        ],
        out_specs=pl.BlockSpec((R, D_IN), lambda i: (i, 0)),
        out_shape=jax.ShapeDtypeStruct((N_NODES, D_IN), jnp.float32),
    )(h, batch3, maxp, sums, counts, conv_w, conv_b)


def kernel(h, edge_index, batch, W, b, conv_w, conv_b):
    batch3 = batch.reshape(NB, 1, R)
    b2d = b.reshape(1, D_HID)
    x, gmax, sums, counts = _run_mm(h, W, b2d, batch3)
    maxp = _run_sc_max(gmax.reshape(NPHYS, D_HID), x, batch)
    h_out = _run_out(h, batch3, maxp, sums, counts, conv_w, conv_b)
    return (h_out, edge_index, batch)
